# Initial kernel scaffold; baseline (speedup 1.0000x reference)
import jax
import jax.numpy as jnp
from jax.experimental import pallas as pl


def kernel(scaled_cosine, golds):
    raise NotImplementedError("write your pallas kernel here")


# speedup vs baseline: 1.1650x; 1.1650x over previous
import functools
import math

import jax
import jax.numpy as jnp
from jax.experimental import pallas as pl
from jax.experimental.pallas import tpu as pltpu

MARGIN_M = 0.5
SCALE_S = 64.0
COS_MM = float(math.cos(MARGIN_M))
SIN_MM = float(math.sin(MARGIN_M))

BR = 8


def _body(x_ref, g_ref, o_ref):
    x = x_ref[...]
    g = g_ref[...]
    c = x * (1.0 / SCALE_S)
    s = jnp.sqrt(jnp.maximum(1.0 - c * c, 0.0))
    m = (c * COS_MM - s * SIN_MM) * SCALE_S
    cols = jax.lax.broadcasted_iota(jnp.int32, x.shape, 1)
    o_ref[...] = jnp.where(cols == g, m, x)


def kernel(scaled_cosine, golds):
    B, V = scaled_cosine.shape
    golds2 = golds.reshape(B, 1).astype(jnp.int32)
    grid = (B // BR,)
    return pl.pallas_call(
        _body,
        grid=grid,
        in_specs=[
            pl.BlockSpec((BR, V), lambda r: (r, 0)),
            pl.BlockSpec((BR, 1), lambda r: (r, 0)),
        ],
        out_specs=pl.BlockSpec((BR, V), lambda r: (r, 0)),
        out_shape=jax.ShapeDtypeStruct((B, V), scaled_cosine.dtype),
    )(scaled_cosine, golds2)


# speedup vs baseline: 1.3374x; 1.1480x over previous
import math

import jax
import jax.numpy as jnp
from jax.experimental import pallas as pl

MARGIN_M = 0.5
SCALE_S = 64.0
COS_MM = float(math.cos(MARGIN_M))
SIN_MM = float(math.sin(MARGIN_M))

BR = 16


def _body(x_ref, g_ref, o_ref):
    x = x_ref[...]
    g = g_ref[...]
    cols = jax.lax.broadcasted_iota(jnp.int32, x.shape, 1)
    mask = cols == g
    pc = jnp.sum(jnp.where(mask, x, 0.0), axis=1, keepdims=True) * (1.0 / SCALE_S)
    s = jnp.sqrt(jnp.maximum(1.0 - pc * pc, 0.0))
    m = (pc * COS_MM - s * SIN_MM) * SCALE_S
    o_ref[...] = jnp.where(mask, m, x)


def kernel(scaled_cosine, golds):
    B, V = scaled_cosine.shape
    golds2 = golds.reshape(B, 1).astype(jnp.int32)
    return pl.pallas_call(
        _body,
        grid=(B // BR,),
        in_specs=[
            pl.BlockSpec((BR, V), lambda r: (r, 0)),
            pl.BlockSpec((BR, 1), lambda r: (r, 0)),
        ],
        out_specs=pl.BlockSpec((BR, V), lambda r: (r, 0)),
        out_shape=jax.ShapeDtypeStruct((B, V), scaled_cosine.dtype),
    )(scaled_cosine, golds2)


# speedup vs baseline: 4.9938x; 3.7339x over previous
import math

import jax
import jax.numpy as jnp
from jax.experimental import pallas as pl

MARGIN_M = 0.5
SCALE_S = 64.0
COS_MM = float(math.cos(MARGIN_M))
SIN_MM = float(math.sin(MARGIN_M))

BRV = 1000


def _body(x_ref, g_ref, o_ref):
    x = x_ref[...]
    g = g_ref[...]
    r = pl.program_id(0)
    rows = jax.lax.broadcasted_iota(jnp.int32, x.shape, 0) + r * BRV
    mask = rows == g
    pc = jnp.sum(jnp.where(mask, x, 0.0), axis=0, keepdims=True) * (1.0 / SCALE_S)
    s = jnp.sqrt(jnp.maximum(1.0 - pc * pc, 0.0))
    m = (pc * COS_MM - s * SIN_MM) * SCALE_S
    o_ref[...] = jnp.where(mask, m, x)


def kernel(scaled_cosine, golds):
    B, V = scaled_cosine.shape
    xt = scaled_cosine.T
    golds2 = golds.reshape(1, B).astype(jnp.int32)
    outt = pl.pallas_call(
        _body,
        grid=(V // BRV,),
        in_specs=[
            pl.BlockSpec((BRV, B), lambda r: (r, 0)),
            pl.BlockSpec((1, B), lambda r: (0, 0)),
        ],
        out_specs=pl.BlockSpec((BRV, B), lambda r: (r, 0)),
        out_shape=jax.ShapeDtypeStruct((V, B), scaled_cosine.dtype),
    )(xt, golds2)
    return outt.T


# speedup vs baseline: 5.0726x; 1.0158x over previous
import math

import jax
import jax.numpy as jnp
from jax.experimental import pallas as pl

MARGIN_M = 0.5
SCALE_S = 64.0
COS_MM = float(math.cos(MARGIN_M))
SIN_MM = float(math.sin(MARGIN_M))

BRV = 2000


def _body(x_ref, g_ref, o_ref):
    x = x_ref[...]
    g = g_ref[...]
    r = pl.program_id(0)
    rows = jax.lax.broadcasted_iota(jnp.int32, x.shape, 0) + r * BRV
    mask = rows == g
    pc = jnp.sum(jnp.where(mask, x, 0.0), axis=0, keepdims=True) * (1.0 / SCALE_S)
    s = jnp.sqrt(jnp.maximum(1.0 - pc * pc, 0.0))
    m = (pc * COS_MM - s * SIN_MM) * SCALE_S
    o_ref[...] = jnp.where(mask, m, x)


def kernel(scaled_cosine, golds):
    B, V = scaled_cosine.shape
    xt = scaled_cosine.T
    golds2 = golds.reshape(1, B).astype(jnp.int32)
    outt = pl.pallas_call(
        _body,
        grid=(V // BRV,),
        in_specs=[
            pl.BlockSpec((BRV, B), lambda r: (r, 0)),
            pl.BlockSpec((1, B), lambda r: (0, 0)),
        ],
        out_specs=pl.BlockSpec((BRV, B), lambda r: (r, 0)),
        out_shape=jax.ShapeDtypeStruct((V, B), scaled_cosine.dtype),
    )(xt, golds2)
    return outt.T
